# baseline (speedup 1.0000x reference)
import functools

import numpy as np
import jax
import jax.numpy as jnp
from jax import lax
from jax.experimental import pallas as pl
from jax.experimental.pallas import tpu as pltpu
from jax.experimental.pallas import tpu_sc as plsc

N = 10000
C = 256
T = 32768
TBITS = 15
TMASK = T - 1
NW = 32
CHUNKS = N // 16
EMPTY = -1
MULT = np.uint32(2654435761)
HEAD, MID, TAIL = 2, 4, 48
IHEAD, IMID = 2, 3
PHEAD, PMID = 3, 3


def _bucket(kv):
    h = kv.astype(jnp.uint32) * MULT
    return (h >> np.uint32(32 - TBITS)).astype(jnp.int32)


def _body(feats_hbm, xs_hbm, ys_hbm, zs_hbm, out_hbm, tab, keys_arr,
          stage_x, stage_y, stage_z, idxg, maskg, idc, rows0, rows1, acc,
          sem0, sem1, sema):
    cid = lax.axis_index("c")
    sid = lax.axis_index("s")
    wid = sid * 2 + cid

    lane = lax.iota(jnp.int32, 16)
    ones = lane < 16
    zeros_i32 = lane * 0

    neg1 = zeros_i32 + EMPTY
    def init_body(v, carry):
        for u in range(16):
            tab[pl.ds(v * 256 + u * 16, 16)] = neg1
        return carry
    lax.fori_loop(0, T // 256, init_body, 0)

    def slot_key(sid_v):
        return plsc.load_gather(keys_arr, [jnp.maximum(sid_v, 0)])

    def ins_round(st, kv, iv):
        p, act = st
        oid = plsc.load_gather(tab, [p])
        empty = act & (oid == EMPTY)
        plsc.store_scatter(tab, [p], iv, mask=empty)
        oid2 = plsc.load_gather(tab, [p])
        k2 = slot_key(oid2)
        havekey = act & (k2 == kv)
        better = havekey & (oid2 > iv)
        plsc.store_scatter(tab, [p], iv, mask=better)
        done = havekey & ~better
        act2 = act & ~done
        adv = act2 & ~havekey
        p2 = jnp.where(adv, (p + 1) & TMASK, p)
        return (p2, act2)

    def load_key(jb, v):
        x = stage_x[pl.ds(v * 16, 16)]
        y = stage_y[pl.ds(v * 16, 16)]
        z = stage_z[pl.ds(v * 16, 16)]
        kv = ((x + 1) * 130 + (y + 1)) * 130 + (z + 1)
        keys_arr[pl.ds(jb * 2000 + v * 16, 16)] = kv
        return kv, lane + (jb * 125 + v) * 16

    def ins_one(kv, iv):
        st = (_bucket(kv), ones)
        for r in range(HEAD):
            st = ins_round(st, kv, iv)
        def mid(s):
            for r in range(MID):
                s = ins_round(s, kv, iv)
            return s
        st = lax.cond(jnp.any(st[1]), mid, lambda s: s, st)
        def tail(s):
            return lax.fori_loop(
                0, TAIL, lambda r, ss: ins_round(ss, kv, iv), s)
        st = lax.cond(jnp.any(st[1]), tail, lambda s: s, st)

    def stage_blk(jb, carry):
        pltpu.sync_copy(xs_hbm.at[pl.ds(jb * 2000, 2000)], stage_x)
        pltpu.sync_copy(ys_hbm.at[pl.ds(jb * 2000, 2000)], stage_y)
        pltpu.sync_copy(zs_hbm.at[pl.ds(jb * 2000, 2000)], stage_z)
        def keyins4(v, c2):
            kvs, ivs, sts = [], [], []
            for u in range(4):
                kvu, ivu = load_key(jb, v * 4 + u)
                kvs.append(kvu)
                ivs.append(ivu)
                sts.append((_bucket(kvu), ones))
            def rounds(ss, n):
                for r in range(n):
                    ss = tuple(ins_round(ss[u], kvs[u], ivs[u])
                               for u in range(4))
                return ss
            sts = rounds(tuple(sts), IHEAD)
            def anyact(ss):
                return (jnp.any(ss[0][1]) | jnp.any(ss[1][1]) |
                        jnp.any(ss[2][1]) | jnp.any(ss[3][1]))
            sts = lax.cond(anyact(sts), lambda s: rounds(s, IMID),
                           lambda s: s, sts)
            def tail(s):
                return lax.fori_loop(0, TAIL, lambda r, ss: rounds(ss, 1), s)
            sts = lax.cond(anyact(sts), tail, lambda s: s, sts)
            return c2
        out = lax.fori_loop(0, 31, keyins4, carry)
        kvl, ivl = load_key(jb, 124)
        ins_one(kvl, ivl)
        return out
    lax.fori_loop(0, 5, stage_blk, 0)

    def probe_round(st, qv):
        p, act, res, fnd = st
        oid = plsc.load_gather(tab, [p])
        okey = slot_key(oid)
        hit = act & (oid >= 0) & (okey == qv)
        stop = hit | (oid == EMPTY)
        res = jnp.where(hit, oid, res)
        fnd = fnd | hit
        act2 = act & ~stop
        p2 = jnp.where(act2, (p + 1) & TMASK, p)
        return (p2, act2, res, fnd)

    def probe(qv):
        st = (_bucket(qv), ones, zeros_i32, lane < 0)
        for r in range(HEAD):
            st = probe_round(st, qv)
        def mid(s):
            for r in range(MID):
                s = probe_round(s, qv)
            return s
        st = lax.cond(jnp.any(st[1]), mid, lambda s: s, st)
        def tail(s):
            return lax.fori_loop(
                0, TAIL, lambda r, ss: probe_round(ss, qv), s)
        st = lax.cond(jnp.any(st[1]), tail, lambda s: s, st)
        return st[2], st[3]

    def chunk_body(j, carry):
        c = j * NW + wid
        @pl.when(c < CHUNKS)
        def _():
            kv = keys_arr[pl.ds(c * 16, 16)]
            ctr, _f = probe(kv)
            idc[...] = ctr
            h_acc = pltpu.async_copy(feats_hbm.at[idc], acc, sema)

            def kdelta(k):
                dx = lax.rem(k, 3) - 1
                dy = lax.rem(lax.div(k, 3), 3) - 1
                dz = lax.div(k, 9) - 1
                return dx * 16900 + dy * 130 + dz
            def append(fnd, res, nh2):
                safe = jnp.where(fnd, res, ctr)
                def yes(nh3):
                    idxg[pl.ds(nh3 * 16, 16)] = safe
                    maskg[pl.ds(nh3 * 16, 16)] = jnp.where(fnd, 1, 0)
                    return nh3 + 1
                return lax.cond(jnp.any(fnd), yes, lambda nh3: nh3, nh2)
            def probe_many(qs):
                sts = [(_bucket(q), ones, zeros_i32, lane < 0) for q in qs]
                nq = len(qs)
                def rounds(ss, n):
                    for r in range(n):
                        ss = tuple(probe_round(ss[u], qs[u])
                                   for u in range(nq))
                    return ss
                def anyact(ss):
                    a = jnp.any(ss[0][1])
                    for u in range(1, nq):
                        a = a | jnp.any(ss[u][1])
                    return a
                sts = rounds(tuple(sts), PHEAD)
                sts = lax.cond(anyact(sts), lambda s: rounds(s, PMID),
                               lambda s: s, sts)
                def tail(s):
                    return lax.fori_loop(
                        0, TAIL, lambda r, ss: rounds(ss, 1), s)
                sts = lax.cond(anyact(sts), tail, lambda s: s, sts)
                return sts
            def scan_q(q, nh):
                i = q * 4
                ks = [i, i + 1, i + 2, i + 3]
                qs = [kv + kdelta(jnp.where(k >= 13, k + 1, k)) for k in ks]
                sts = probe_many(qs)
                for u in range(4):
                    nh = append(sts[u][3], sts[u][2], nh)
                return nh
            nh = lax.fori_loop(0, 6, scan_q, 0)
            stl = probe_many([kv + kdelta(25), kv + kdelta(26)])
            nh = append(stl[0][3], stl[0][2], nh)
            nh = append(stl[1][3], stl[1][2], nh)

            bufs = (rows0, rows1)
            sems = (sem0, sem1)
            def fire(i, buf, sem):
                pltpu.async_copy(
                    feats_hbm.at[idxg.at[pl.ds(i * 16, 16)]], buf, sem)
            @pl.when(nh > 0)
            def _():
                fire(0, rows0, sem0)
            @pl.when(nh > 1)
            def _():
                fire(1, rows1, sem1)
            h_acc.wait()

            def fold_from(buf, gi):
                mv = maskg[pl.ds(gi * 16, 16)]
                for p in range(16):
                    @pl.when(jnp.any((mv != 0) & (lane == p)))
                    def _(p=p):
                        def fold(cb, c3):
                            sl = pl.ds(cb * 16, 16)
                            acc[p, sl] = jnp.maximum(acc[p, sl], buf[p, sl])
                            return c3
                        lax.fori_loop(0, C // 16, fold, 0)
            def gloop(i, carry):
                def go(buf, sem):
                    pltpu.make_async_copy(
                        feats_hbm.at[idc], buf, sem).wait()
                    fold_from(buf, i)
                    @pl.when(i + 2 < nh)
                    def _():
                        fire(i + 2, buf, sem)
                    return 0
                lax.cond(lax.rem(i, 2) == 0,
                         lambda: go(rows0, sem0),
                         lambda: go(rows1, sem1))
                return carry
            lax.fori_loop(0, nh, gloop, 0)
            pltpu.sync_copy(acc, out_hbm.at[pl.ds(c * 16, 16)])
        return carry
    lax.fori_loop(0, (CHUNKS + NW - 1) // NW, chunk_body, 0)


@functools.partial(jax.jit, static_argnums=())
def _pool(feats, xs, ys, zs):
    mesh = plsc.VectorSubcoreMesh(
        core_axis_name="c", subcore_axis_name="s", num_cores=2,
        num_subcores=16)
    f = pl.kernel(
        _body,
        out_type=jax.ShapeDtypeStruct((N, C), jnp.float32),
        mesh=mesh,
        compiler_params=pltpu.CompilerParams(needs_layout_passes=False),
        scratch_types=[
            pltpu.VMEM((T,), jnp.int32),
            pltpu.VMEM((N,), jnp.int32),
            pltpu.VMEM((2000,), jnp.int32),
            pltpu.VMEM((2000,), jnp.int32),
            pltpu.VMEM((2000,), jnp.int32),
            pltpu.VMEM((27 * 16,), jnp.int32),
            pltpu.VMEM((27 * 16,), jnp.int32),
            pltpu.VMEM((16,), jnp.int32),
            pltpu.VMEM((16, C), jnp.float32),
            pltpu.VMEM((16, C), jnp.float32),
            pltpu.VMEM((16, C), jnp.float32),
            pltpu.SemaphoreType.DMA,
            pltpu.SemaphoreType.DMA,
            pltpu.SemaphoreType.DMA,
        ],
    )
    return f(feats, xs, ys, zs)


def kernel(feats, coords):
    return _pool(feats, coords[:, 0], coords[:, 1], coords[:, 2])


# speedup vs baseline: 1.5602x; 1.5602x over previous
import functools

import numpy as np
import jax
import jax.numpy as jnp
from jax import lax
from jax.experimental import pallas as pl
from jax.experimental.pallas import tpu as pltpu
from jax.experimental.pallas import tpu_sc as plsc

N = 10000
C = 256
T = 65536
TBITS = 16
TMASK = T - 1
NW = 32
CHUNKS = N // 16
EMPTY = -1
MULT = np.uint32(2654435761)
HEAD, MID, TAIL = 2, 4, 48
IHEAD, IMID = 2, 3
PHEAD, PMID = 3, 3


def _bucket(kv):
    h = kv.astype(jnp.uint32) * MULT
    return (h >> np.uint32(32 - TBITS)).astype(jnp.int32)


def _body(feats_hbm, xs_hbm, ys_hbm, zs_hbm, out_hbm, tab, keys_arr,
          stage_x, stage_y, stage_z, idxg, maskg, idc, rows0, rows1, acc,
          sem0, sem1, sema):
    cid = lax.axis_index("c")
    sid = lax.axis_index("s")
    wid = sid * 2 + cid

    lane = lax.iota(jnp.int32, 16)
    ones = lane < 16
    zeros_i32 = lane * 0

    neg1 = zeros_i32 + EMPTY
    def init_body(v, carry):
        for u in range(16):
            tab[pl.ds(v * 256 + u * 16, 16)] = neg1
        return carry
    lax.fori_loop(0, T // 256, init_body, 0)

    def slot_key(sid_v):
        return plsc.load_gather(keys_arr, [jnp.maximum(sid_v, 0)])

    def ins_round(st, kv, iv):
        p, act = st
        oid = plsc.load_gather(tab, [p])
        empty = act & (oid == EMPTY)
        plsc.store_scatter(tab, [p], iv, mask=empty)
        oid2 = plsc.load_gather(tab, [p])
        k2 = slot_key(oid2)
        havekey = act & (k2 == kv)
        better = havekey & (oid2 > iv)
        plsc.store_scatter(tab, [p], iv, mask=better)
        done = havekey & ~better
        act2 = act & ~done
        adv = act2 & ~havekey
        p2 = jnp.where(adv, (p + 1) & TMASK, p)
        return (p2, act2)

    def load_key(jb, v):
        x = stage_x[pl.ds(v * 16, 16)]
        y = stage_y[pl.ds(v * 16, 16)]
        z = stage_z[pl.ds(v * 16, 16)]
        kv = ((x + 1) * 130 + (y + 1)) * 130 + (z + 1)
        keys_arr[pl.ds(jb * 2000 + v * 16, 16)] = kv
        return kv, lane + (jb * 125 + v) * 16

    def ins_one(kv, iv):
        st = (_bucket(kv), ones)
        for r in range(HEAD):
            st = ins_round(st, kv, iv)
        def mid(s):
            for r in range(MID):
                s = ins_round(s, kv, iv)
            return s
        st = lax.cond(jnp.any(st[1]), mid, lambda s: s, st)
        def tail(s):
            return lax.fori_loop(
                0, TAIL, lambda r, ss: ins_round(ss, kv, iv), s)
        st = lax.cond(jnp.any(st[1]), tail, lambda s: s, st)

    def stage_blk(jb, carry):
        pltpu.sync_copy(xs_hbm.at[pl.ds(jb * 2000, 2000)], stage_x)
        pltpu.sync_copy(ys_hbm.at[pl.ds(jb * 2000, 2000)], stage_y)
        pltpu.sync_copy(zs_hbm.at[pl.ds(jb * 2000, 2000)], stage_z)
        def keyins4(v, c2):
            kvs, ivs, sts = [], [], []
            for u in range(4):
                kvu, ivu = load_key(jb, v * 4 + u)
                kvs.append(kvu)
                ivs.append(ivu)
                sts.append((_bucket(kvu), ones))
            def rounds(ss, n):
                for r in range(n):
                    ss = tuple(ins_round(ss[u], kvs[u], ivs[u])
                               for u in range(4))
                return ss
            sts = rounds(tuple(sts), IHEAD)
            def anyact(ss):
                return (jnp.any(ss[0][1]) | jnp.any(ss[1][1]) |
                        jnp.any(ss[2][1]) | jnp.any(ss[3][1]))
            sts = lax.cond(anyact(sts), lambda s: rounds(s, IMID),
                           lambda s: s, sts)
            def tail(s):
                return lax.fori_loop(0, TAIL, lambda r, ss: rounds(ss, 1), s)
            sts = lax.cond(anyact(sts), tail, lambda s: s, sts)
            return c2
        out = lax.fori_loop(0, 31, keyins4, carry)
        kvl, ivl = load_key(jb, 124)
        ins_one(kvl, ivl)
        return out
    lax.fori_loop(0, 5, stage_blk, 0)

    def probe_round(st, qv):
        p, act, res, fnd = st
        oid = plsc.load_gather(tab, [p])
        okey = slot_key(oid)
        hit = act & (oid >= 0) & (okey == qv)
        stop = hit | (oid == EMPTY)
        res = jnp.where(hit, oid, res)
        fnd = fnd | hit
        act2 = act & ~stop
        p2 = jnp.where(act2, (p + 1) & TMASK, p)
        return (p2, act2, res, fnd)

    def probe(qv):
        st = (_bucket(qv), ones, zeros_i32, lane < 0)
        for r in range(HEAD):
            st = probe_round(st, qv)
        def mid(s):
            for r in range(MID):
                s = probe_round(s, qv)
            return s
        st = lax.cond(jnp.any(st[1]), mid, lambda s: s, st)
        def tail(s):
            return lax.fori_loop(
                0, TAIL, lambda r, ss: probe_round(ss, qv), s)
        st = lax.cond(jnp.any(st[1]), tail, lambda s: s, st)
        return st[2], st[3]

    def chunk_body(j, carry):
        c = j * NW + wid
        @pl.when(c < CHUNKS)
        def _():
            kv = keys_arr[pl.ds(c * 16, 16)]
            ctr, _f = probe(kv)
            idc[...] = ctr
            h_acc = pltpu.async_copy(feats_hbm.at[idc], acc, sema)

            def kdelta(k):
                dx = lax.rem(k, 3) - 1
                dy = lax.rem(lax.div(k, 3), 3) - 1
                dz = lax.div(k, 9) - 1
                return dx * 16900 + dy * 130 + dz
            def append(fnd, res, nh2):
                safe = jnp.where(fnd, res, ctr)
                def yes(nh3):
                    idxg[pl.ds(nh3 * 16, 16)] = safe
                    maskg[pl.ds(nh3 * 16, 16)] = jnp.where(fnd, 1, 0)
                    return nh3 + 1
                return lax.cond(jnp.any(fnd), yes, lambda nh3: nh3, nh2)
            def probe_many(qs):
                sts = [(_bucket(q), ones, zeros_i32, lane < 0) for q in qs]
                nq = len(qs)
                def rounds(ss, n):
                    for r in range(n):
                        ss = tuple(probe_round(ss[u], qs[u])
                                   for u in range(nq))
                    return ss
                def anyact(ss):
                    a = jnp.any(ss[0][1])
                    for u in range(1, nq):
                        a = a | jnp.any(ss[u][1])
                    return a
                sts = rounds(tuple(sts), PHEAD)
                sts = lax.cond(anyact(sts), lambda s: rounds(s, PMID),
                               lambda s: s, sts)
                def tail(s):
                    return lax.fori_loop(
                        0, TAIL, lambda r, ss: rounds(ss, 1), s)
                sts = lax.cond(anyact(sts), tail, lambda s: s, sts)
                return sts
            def scan_q(q, nh):
                i = q * 4
                ks = [i, i + 1, i + 2, i + 3]
                qs = [kv + kdelta(jnp.where(k >= 13, k + 1, k)) for k in ks]
                sts = probe_many(qs)
                for u in range(4):
                    nh = append(sts[u][3], sts[u][2], nh)
                return nh
            nh = lax.fori_loop(0, 6, scan_q, 0)
            stl = probe_many([kv + kdelta(25), kv + kdelta(26)])
            nh = append(stl[0][3], stl[0][2], nh)
            nh = append(stl[1][3], stl[1][2], nh)

            bufs = (rows0, rows1)
            sems = (sem0, sem1)
            def fire(i, buf, sem):
                pltpu.async_copy(
                    feats_hbm.at[idxg.at[pl.ds(i * 16, 16)]], buf, sem)
            @pl.when(nh > 0)
            def _():
                fire(0, rows0, sem0)
            @pl.when(nh > 1)
            def _():
                fire(1, rows1, sem1)
            h_acc.wait()

            def fold_from(buf, gi):
                mv = maskg[pl.ds(gi * 16, 16)]
                for p in range(16):
                    @pl.when(jnp.any((mv != 0) & (lane == p)))
                    def _(p=p):
                        def fold(cb, c3):
                            sl = pl.ds(cb * 16, 16)
                            acc[p, sl] = jnp.maximum(acc[p, sl], buf[p, sl])
                            return c3
                        lax.fori_loop(0, C // 16, fold, 0)
            def gloop(i, carry):
                def go(buf, sem):
                    pltpu.make_async_copy(
                        feats_hbm.at[idc], buf, sem).wait()
                    fold_from(buf, i)
                    @pl.when(i + 2 < nh)
                    def _():
                        fire(i + 2, buf, sem)
                    return 0
                lax.cond(lax.rem(i, 2) == 0,
                         lambda: go(rows0, sem0),
                         lambda: go(rows1, sem1))
                return carry
            lax.fori_loop(0, nh, gloop, 0)
            pltpu.sync_copy(acc, out_hbm.at[pl.ds(c * 16, 16)])
        return carry
    lax.fori_loop(0, (CHUNKS + NW - 1) // NW, chunk_body, 0)


@functools.partial(jax.jit, static_argnums=())
def _pool(feats, xs, ys, zs):
    mesh = plsc.VectorSubcoreMesh(
        core_axis_name="c", subcore_axis_name="s", num_cores=2,
        num_subcores=16)
    f = pl.kernel(
        _body,
        out_type=jax.ShapeDtypeStruct((N, C), jnp.float32),
        mesh=mesh,
        compiler_params=pltpu.CompilerParams(needs_layout_passes=False),
        scratch_types=[
            pltpu.VMEM((T,), jnp.int32),
            pltpu.VMEM((N,), jnp.int32),
            pltpu.VMEM((2000,), jnp.int32),
            pltpu.VMEM((2000,), jnp.int32),
            pltpu.VMEM((2000,), jnp.int32),
            pltpu.VMEM((27 * 16,), jnp.int32),
            pltpu.VMEM((27 * 16,), jnp.int32),
            pltpu.VMEM((16,), jnp.int32),
            pltpu.VMEM((16, C), jnp.float32),
            pltpu.VMEM((16, C), jnp.float32),
            pltpu.VMEM((16, C), jnp.float32),
            pltpu.SemaphoreType.DMA,
            pltpu.SemaphoreType.DMA,
            pltpu.SemaphoreType.DMA,
        ],
    )
    return f(feats, xs, ys, zs)


def kernel(feats, coords):
    return _pool(feats, coords[:, 0], coords[:, 1], coords[:, 2])
